# Initial kernel scaffold; baseline (speedup 1.0000x reference)
import jax
import jax.numpy as jnp
from jax.experimental import pallas as pl


def kernel(ctx_signal_enc_nchw, density_enc_nchw, w0, b0, dww, dwb, lng, lnb, pw1w, pw1b, pw2w, pw2b, wf_mu, bf_mu, wf_ls, bf_ls):
    raise NotImplementedError("write your pallas kernel here")


# speedup vs baseline: 1.1297x; 1.1297x over previous
import functools

import jax
import jax.numpy as jnp
from jax import lax
from jax.experimental import pallas as pl
from jax.experimental.pallas import tpu as pltpu

N_BLOCKS = 5
KS = 7
PAD = KS // 2
W_OFF = 8
RC = 8
OUT_LANES = 16
SIG_OFF = 8


def _gelu_tanh(x):
    t = x * x
    u = x * (0.7978845608028654 + 0.035677408136300125 * t)
    return (0.5 * x) * (1.0 + jnp.tanh(u))


def _decoder_body(H, W,
                  den_ref, ctx_ref,
                  w0d_ref, w0c_ref, b0_ref, mavg_ref,
                  dww_ref, dwb_ref,
                  pw1w_ref, pw1b_ref, pw2w_ref, pw2b_ref,
                  wf_ref, bf_ref, sel_ref,
                  out_ref,
                  pad_a, pad_b):
    C = den_ref.shape[3]
    HP, WP, _ = pad_a.shape
    n_chunks = H // RC
    MC = RC * W

    for p in (pad_a, pad_b):
        p[0:PAD, :, :] = jnp.zeros((PAD, WP, C), jnp.float32)
        p[PAD + H:HP, :, :] = jnp.zeros((HP - PAD - H, WP, C), jnp.float32)
        p[PAD:PAD + H, 0:W_OFF, :] = jnp.zeros((H, W_OFF, C), jnp.float32)
        p[PAD:PAD + H, W_OFF + W:WP, :] = jnp.zeros((H, WP - W_OFF - W, C),
                                                    jnp.float32)

    def in_body(c, carry):
        h0 = pl.multiple_of(c * RC, RC)
        d = den_ref[0, pl.ds(h0, RC), :, :].reshape(MC, C)
        cx = ctx_ref[0, pl.ds(h0, RC), :, :].reshape(MC, C)
        x0 = jnp.dot(d, w0d_ref[...], preferred_element_type=jnp.float32)
        x0 = x0 + jnp.dot(cx, w0c_ref[...], preferred_element_type=jnp.float32)
        x0 = x0 + b0_ref[0]
        pad_a[pl.ds(PAD + h0, RC), W_OFF:W_OFF + W, :] = x0.reshape(RC, W, C)
        return carry

    lax.fori_loop(0, n_chunks, in_body, 0, unroll=2)

    for blk in range(N_BLOCKS):
        src, dst = (pad_a, pad_b) if blk % 2 == 0 else (pad_b, pad_a)

        def chunk_body(c, carry, blk=blk, src=src, dst=dst):
            h0 = pl.multiple_of(c * RC, RC)
            wtaps = dww_ref[blk]

            acc = jnp.zeros((RC, W, C), jnp.float32)
            center = None
            for dw in range(KS):
                s = src[pl.ds(h0, RC + KS - 1),
                        W_OFF - PAD + dw:W_OFF - PAD + dw + W, :]
                for dh in range(KS):
                    acc = acc + s[dh:dh + RC] * wtaps[dh, dw]
                if dw == PAD:
                    center = s[PAD:PAD + RC]
            y = acc + dwb_ref[blk]

            yf = y.reshape(MC, C)
            stat_in = jnp.concatenate([yf, yf * yf], axis=0).astype(jnp.bfloat16)
            stats = jnp.dot(stat_in, mavg_ref[...],
                            preferred_element_type=jnp.float32)
            mean = stats[:MC]
            var = stats[MC:] - mean * mean
            yn = (yf - mean) * lax.rsqrt(var + 1e-6)

            hg = jnp.dot(yn.astype(jnp.bfloat16), pw1w_ref[blk],
                         preferred_element_type=jnp.float32) + pw1b_ref[blk]
            hg = _gelu_tanh(hg)
            y2 = jnp.dot(hg.astype(jnp.bfloat16), pw2w_ref[blk],
                         preferred_element_type=jnp.float32) + pw2b_ref[blk]

            dst[pl.ds(PAD + h0, RC), W_OFF:W_OFF + W, :] = (
                center + y2.reshape(RC, W, C))
            return carry

        lax.fori_loop(0, n_chunks, chunk_body, 0, unroll=2)

    final = pad_b if N_BLOCKS % 2 == 1 else pad_a

    def head_body(c, carry):
        h0 = pl.multiple_of(c * RC, RC)
        xc = final[pl.ds(PAD + h0, RC), W_OFF:W_OFF + W, :].reshape(MC, C)
        z = jnp.dot(xc.astype(jnp.bfloat16), wf_ref[...],
                    preferred_element_type=jnp.float32) + bf_ref[0]
        sp = jnp.maximum(z, 0.0) + jnp.log(1.0 + jnp.exp(-jnp.abs(z)))
        zout = jnp.where(sel_ref[0] > 0.0, 0.1 + 0.9 * sp, z)
        out_ref[0, pl.ds(pl.multiple_of(h0 * W, MC), MC), :] = zout
        return carry

    lax.fori_loop(0, n_chunks, head_body, 0, unroll=2)


def kernel(ctx_signal_enc_nchw, density_enc_nchw, w0, b0, dww, dwb, lng, lnb,
           pw1w, pw1b, pw2w, pw2b, wf_mu, bf_mu, wf_ls, bf_ls):
    B, L, H, W = ctx_signal_enc_nchw.shape
    n_in = wf_mu.shape[1]

    den = jnp.transpose(density_enc_nchw, (0, 2, 3, 1)).astype(jnp.bfloat16)
    ctx = jnp.transpose(ctx_signal_enc_nchw, (0, 2, 3, 1)).astype(jnp.bfloat16)

    w0d = w0[:L].astype(jnp.bfloat16)
    w0c = w0[L:].astype(jnp.bfloat16)
    mavg = jnp.full((L, L), 1.0 / L, jnp.bfloat16)
    pw1w_eff = (lng[:, :, None] * pw1w).astype(jnp.bfloat16)
    pw1b_eff = jnp.einsum('nl,nlh->nh', lnb, pw1w) + pw1b
    pw2w_b = pw2w.astype(jnp.bfloat16)

    wf = jnp.zeros((L, OUT_LANES), jnp.float32)
    wf = wf.at[:, :n_in].set(wf_mu)
    wf = wf.at[:, SIG_OFF:SIG_OFF + n_in].set(wf_ls)
    wf = wf.astype(jnp.bfloat16)
    bf = jnp.zeros((1, OUT_LANES), jnp.float32)
    bf = bf.at[:, :n_in].set(bf_mu)
    bf = bf.at[:, SIG_OFF:SIG_OFF + n_in].set(bf_ls)
    sel = jnp.zeros((1, OUT_LANES), jnp.float32).at[:, SIG_OFF:SIG_OFF + n_in].set(1.0)

    weight_args = (w0d, w0c, b0, mavg, dww, dwb,
                   pw1w_eff, pw1b_eff, pw2w_b, pw2b, wf, bf, sel)

    def full_spec(a):
        return pl.BlockSpec(a.shape, lambda b, _n=a.ndim: (0,) * _n)

    in_specs = [pl.BlockSpec((1, H, W, L), lambda b: (b, 0, 0, 0)),
                pl.BlockSpec((1, H, W, L), lambda b: (b, 0, 0, 0))]
    in_specs += [full_spec(w) for w in weight_args]
    out_specs = pl.BlockSpec((1, H * W, OUT_LANES), lambda b: (b, 0, 0))

    HP = H + 2 * PAD
    WP = W_OFF + W + 8

    body = functools.partial(_decoder_body, H, W)
    out = pl.pallas_call(
        body,
        out_shape=jax.ShapeDtypeStruct((B, H * W, OUT_LANES), jnp.float32),
        grid_spec=pltpu.PrefetchScalarGridSpec(
            num_scalar_prefetch=0,
            grid=(B,),
            in_specs=in_specs,
            out_specs=out_specs,
            scratch_shapes=[pltpu.VMEM((HP, WP, L), jnp.float32),
                            pltpu.VMEM((HP, WP, L), jnp.float32)]),
        compiler_params=pltpu.CompilerParams(
            dimension_semantics=("parallel",),
            vmem_limit_bytes=48 << 20),
    )(den, ctx, *weight_args)

    mu = out[:, :, :n_in]
    sigma = out[:, :, SIG_OFF:SIG_OFF + n_in]
    return mu, sigma
